# baseline (speedup 1.0000x reference)
import functools

import jax
import jax.numpy as jnp
from jax import lax
from jax.experimental import pallas as pl
from jax.experimental.pallas import tpu as pltpu
from jax.experimental.pallas import tpu_sc as plsc

N = 10000
E = 320000
F = 128
H = 64
NEXP = 4
NCLS = 10
NGRAPH = 128

NC, NS = 2, 16
EPC = E // NC
EPW = EPC // NS
SUB = 80
CHUNK = 400
NSUB = CHUNK // SUB
NCHUNK = EPW // CHUNK
SW = 624
ZR = 24
NZ = SW // ZR
TAIL = N - NS * SW

_f32 = jnp.float32
_i32 = jnp.int32


def _mesh():
    return plsc.VectorSubcoreMesh(core_axis_name="c", subcore_axis_name="s",
                                  num_cores=NC, num_subcores=NS)


def _maybe(cond, fn):
    if isinstance(cond, bool):
        if cond:
            fn()
    else:
        pl.when(cond)(fn)


def _zero_acc(zbuf, acc, sid, d):
    def zb(i, c):
        for j in range(d // 16):
            zbuf[i, pl.ds(j * 16, 16)] = jnp.zeros((16,), _f32)
        return c

    lax.fori_loop(0, ZR, zb, 0)

    def zc(i, c):
        pltpu.sync_copy(zbuf, acc.at[pl.ds(sid * SW + i * ZR, ZR)])
        return c

    lax.fori_loop(0, NZ, zc, 0)

    @pl.when(sid == NS - 1)
    def _():
        pltpu.sync_copy(zbuf.at[pl.ds(0, TAIL)], acc.at[pl.ds(NS * SW, TAIL)])


def _make_segsum(d, weighted):
    NSL = 3
    scratch = (
        [pltpu.VMEM((NSUB, SUB), _i32) for _ in range(NSL)]
        + [pltpu.VMEM((NSUB, SUB), _i32) for _ in range(NSL)]
        + [pltpu.VMEM((CHUNK,), _f32) for _ in range(NSL)]
        + [pltpu.VMEM((CHUNK, d), _f32) for _ in range(NSL)]
        + [pltpu.VMEM((ZR, d), _f32)]
        + [pltpu.VMEM_SHARED((N, d), _f32)]
        + [pltpu.SemaphoreType.DMA for _ in range(3 * NSL)]
    )

    def body(*refs):
        nin = 4 if weighted else 3
        h_hbm, src_hbm, dst_hbm = refs[0], refs[1], refs[2]
        w_hbm = refs[3] if weighted else None
        out_hbm = refs[nin]
        r = list(refs[nin + 1:])
        sidx = r[0:NSL]
        didx = r[NSL:2 * NSL]
        wbuf = r[2 * NSL:3 * NSL]
        rows = r[3 * NSL:4 * NSL]
        zbuf = r[4 * NSL]
        acc = r[4 * NSL + 1]
        lsem = r[4 * NSL + 2:4 * NSL + 2 + NSL]
        gsem = r[4 * NSL + 2 + NSL:4 * NSL + 2 + 2 * NSL]
        ssem = r[4 * NSL + 2 + 2 * NSL:]
        cid = lax.axis_index("c")
        sid = lax.axis_index("s")
        base = cid * EPC + sid * EPW

        _zero_acc(zbuf, acc, sid, d)
        plsc.subcore_barrier()

        def load_copies(g, s, f):
            off = base + g * CHUNK
            for j in range(NSUB):
                f(src_hbm.at[pl.ds(off + j * SUB, SUB)], sidx[s].at[j],
                  lsem[s])
                f(dst_hbm.at[pl.ds(off + j * SUB, SUB)], didx[s].at[j],
                  lsem[s])
            if weighted:
                f(w_hbm.at[pl.ds(off, CHUNK)], wbuf[s], lsem[s])

        def gather_copies(s, f):
            for j in range(NSUB):
                f(h_hbm.at[sidx[s].at[j]], rows[s].at[pl.ds(j * SUB, SUB)],
                  gsem[s])

        def scatter_copies(s, f):
            for j in range(NSUB):
                f(rows[s].at[pl.ds(j * SUB, SUB)], acc.at[didx[s].at[j]],
                  ssem[s], add=True)

        def fire(*a, **k):
            pltpu.async_copy(*a, **k)

        def drain(*a, **k):
            pltpu.make_async_copy(*a).wait()

        def mul(s):
            if not weighted:
                return

            def mul16(t, c):
                wreg = wbuf[s][pl.ds(t * 16, 16)]
                for k in range(16):
                    e = t * 16 + k
                    bc = jnp.full((16,), wreg[k], _f32)
                    for j in range(d // 16):
                        rows[s][e, pl.ds(j * 16, 16)] = (
                            rows[s][e, pl.ds(j * 16, 16)] * bc)
                return c

            lax.fori_loop(0, CHUNK // 16, mul16, 0)

        load_copies(0, 0, fire)
        load_copies(0, 0, drain)
        gather_copies(0, fire)

        def chunk_step(g, s):
            nxt = (s + 1) % NSL

            def pre():
                _maybe(g + 1 >= NSL, lambda: scatter_copies(nxt, drain))
                load_copies(g + 1, nxt, fire)

            _maybe(g + 1 < NCHUNK, pre)

            gather_copies(s, drain)
            mul(s)
            scatter_copies(s, fire)

            def post():
                load_copies(g + 1, nxt, drain)
                gather_copies(nxt, fire)

            _maybe(g + 1 < NCHUNK, post)

        def step(t, carry):
            for k in range(NSL):
                chunk_step(NSL * t + k, k)
            return carry

        lax.fori_loop(0, NCHUNK // NSL, step, 0)
        gl = NCHUNK - 1
        chunk_step(gl, gl % NSL)
        scatter_copies((gl - 2) % NSL, drain)
        scatter_copies((gl - 1) % NSL, drain)
        scatter_copies(gl % NSL, drain)

        plsc.subcore_barrier()
        pltpu.sync_copy(acc.at[pl.ds(sid * SW, SW)],
                        out_hbm.at[cid, pl.ds(sid * SW, SW)])

        @pl.when(sid == NS - 1)
        def _():
            pltpu.sync_copy(acc.at[pl.ds(NS * SW, TAIL)],
                            out_hbm.at[cid, pl.ds(NS * SW, TAIL)])

    return pl.kernel(
        body,
        out_type=jax.ShapeDtypeStruct((2, N, d), _f32),
        mesh=_mesh(),
        scratch_types=scratch,
        compiler_params=pltpu.CompilerParams(use_tc_tiling_on_sc=False),
    )


def _make_gather_rows(d):
    NSL = 3
    scratch = (
        [pltpu.VMEM((NSUB, SUB), _i32) for _ in range(NSL)]
        + [pltpu.VMEM((CHUNK, d), _f32) for _ in range(NSL)]
        + [pltpu.SemaphoreType.DMA for _ in range(3 * NSL)]
    )

    def body(h_hbm, idx_hbm, out_hbm, *r):
        sidx = r[0:NSL]
        rows = r[NSL:2 * NSL]
        lsem = r[2 * NSL:3 * NSL]
        gsem = r[3 * NSL:4 * NSL]
        osem = r[4 * NSL:5 * NSL]
        cid = lax.axis_index("c")
        sid = lax.axis_index("s")
        base = cid * EPC + sid * EPW

        def load_copies(g, s, f):
            off = base + g * CHUNK
            for j in range(NSUB):
                f(idx_hbm.at[pl.ds(off + j * SUB, SUB)], sidx[s].at[j],
                  lsem[s])

        def gather_copies(s, f):
            for j in range(NSUB):
                f(h_hbm.at[sidx[s].at[j]], rows[s].at[pl.ds(j * SUB, SUB)],
                  gsem[s])

        def out_copies(g, s, f):
            f(rows[s], out_hbm.at[pl.ds(base + g * CHUNK, CHUNK)], osem[s])

        def fire(*a, **k):
            pltpu.async_copy(*a, **k)

        def drain(*a, **k):
            pltpu.make_async_copy(*a, **k).wait()

        load_copies(0, 0, fire)
        load_copies(0, 0, drain)
        gather_copies(0, fire)

        def chunk_step(g, s):
            nxt = (s + 1) % NSL

            def pre():
                _maybe(g + 1 >= NSL, lambda: out_copies(g - 2, nxt, drain))
                load_copies(g + 1, nxt, fire)

            _maybe(g + 1 < NCHUNK, pre)

            gather_copies(s, drain)
            out_copies(g, s, fire)

            def post():
                load_copies(g + 1, nxt, drain)
                gather_copies(nxt, fire)

            _maybe(g + 1 < NCHUNK, post)

        def step(t, carry):
            for k in range(NSL):
                chunk_step(NSL * t + k, k)
            return carry

        lax.fori_loop(0, NCHUNK // NSL, step, 0)
        gl = NCHUNK - 1
        chunk_step(gl, gl % NSL)
        out_copies(gl - 2, (gl - 2) % NSL, drain)
        out_copies(gl - 1, (gl - 1) % NSL, drain)
        out_copies(gl, gl % NSL, drain)

    return pl.kernel(
        body,
        out_type=jax.ShapeDtypeStruct((E, d), _f32),
        mesh=_mesh(),
        scratch_types=scratch,
        compiler_params=pltpu.CompilerParams(use_tc_tiling_on_sc=False),
    )


_BN = 1000
_NB = N // _BN
_BE = 2000


def _full(shape):
    return pl.BlockSpec(shape, lambda *_: tuple(0 for _ in shape))


def _gin_mlp_kernel(nagg, h_ref, *refs):
    aggs = refs[:nagg]
    scale_ref, w1_ref, b1_ref, w2_ref, b2_ref, o_ref = refs[nagg:]
    hs = h_ref[...] * scale_ref[0, 0]
    if nagg == 1:
        a = hs + aggs[0][0] + aggs[0][1]
    else:
        a = jnp.concatenate(
            [hs[:, :H] + aggs[0][0] + aggs[0][1],
             hs[:, H:] + aggs[1][0] + aggs[1][1]], axis=1)
    t = jnp.maximum(
        jnp.dot(a, w1_ref[...], preferred_element_type=_f32) + b1_ref[...], 0.0)
    o = jnp.dot(t, w2_ref[...], preferred_element_type=_f32) + b2_ref[...]
    o_ref[...] = jnp.maximum(o, 0.0)


def _gin_layer(h, aggs, p):
    din = h.shape[1]
    scale = (1.0 + p["eps"]).reshape(1, 1).astype(_f32)
    return pl.pallas_call(
        functools.partial(_gin_mlp_kernel, len(aggs)),
        grid=(_NB,),
        in_specs=[
            pl.BlockSpec((_BN, din), lambda i: (i, 0)),
        ] + [
            pl.BlockSpec((2, _BN, H), lambda i: (0, i, 0)) for _ in aggs
        ] + [
            _full((1, 1)),
            _full((din, H)),
            _full((1, H)),
            _full((H, H)),
            _full((1, H)),
        ],
        out_specs=pl.BlockSpec((_BN, H), lambda i: (i, 0)),
        out_shape=jax.ShapeDtypeStruct((N, H), _f32),
    )(h, *aggs, scale, p["W1"], p["b1"].reshape(1, H), p["W2"],
      p["b2"].reshape(1, H))


def _gin_mlp4_kernel(nagg, h_ref, *refs):
    aggs = refs[:nagg]
    scale_ref, w1_ref, b1_ref, w2_ref, b2_ref, o_ref = refs[nagg:]
    hs = h_ref[0] * scale_ref[0, 0]
    if nagg == 1:
        a = hs + aggs[0][0, 0] + aggs[0][0, 1]
    else:
        a = jnp.concatenate(
            [hs[:, :H] + aggs[0][0, 0] + aggs[0][0, 1],
             hs[:, H:] + aggs[1][0, 0] + aggs[1][0, 1]], axis=1)
    t = jnp.maximum(
        jnp.dot(a, w1_ref[...], preferred_element_type=_f32) + b1_ref[...], 0.0)
    o = jnp.dot(t, w2_ref[...], preferred_element_type=_f32) + b2_ref[...]
    o_ref[0] = jnp.maximum(o, 0.0)


def _gin_layer4(hc, agg4s, p):
    din = hc.shape[2]
    scale = (1.0 + p["eps"]).reshape(1, 1).astype(_f32)
    return pl.pallas_call(
        functools.partial(_gin_mlp4_kernel, len(agg4s)),
        grid=(NEXP, _NB),
        in_specs=[
            pl.BlockSpec((1, _BN, din), lambda e, i: (e, i, 0)),
        ] + [
            pl.BlockSpec((1, 2, _BN, H), lambda e, i: (e, 0, i, 0))
            for _ in agg4s
        ] + [
            _full((1, 1)),
            _full((din, H)),
            _full((1, H)),
            _full((H, H)),
            _full((1, H)),
        ],
        out_specs=pl.BlockSpec((1, _BN, H), lambda e, i: (e, i, 0)),
        out_shape=jax.ShapeDtypeStruct((NEXP, N, H), _f32),
    )(hc, *agg4s, scale, p["W1"], p["b1"].reshape(1, H), p["W2"],
      p["b2"].reshape(1, H))


def _prep_kernel(x_ref, h_ref, nw1, nb1, nw2, nb2, fw1, fb1, fw2, fb2,
                 xm_ref, nm_ref):
    h = h_ref[...]
    x = x_ref[...]
    for e in range(NEXP):
        t = jnp.maximum(
            jnp.dot(h, nw1[e], preferred_element_type=_f32) + nb1[e], 0.0)
        nm = jax.nn.sigmoid(
            jnp.dot(t, nw2[e], preferred_element_type=_f32) + nb2[e])
        t2 = jnp.maximum(
            jnp.dot(h, fw1[e], preferred_element_type=_f32) + fb1[e], 0.0)
        fm = jax.nn.sigmoid(
            jnp.dot(t2, fw2[e], preferred_element_type=_f32) + fb2[e])
        xm_ref[e] = x * fm * nm
        nm_ref[:, e:e + 1] = nm


def _prep(x, h, nmps, fmps):
    nw1 = jnp.stack([p["W1"] for p in nmps])
    nb1 = jnp.stack([p["b1"].reshape(1, H) for p in nmps])
    nw2 = jnp.stack([p["W2"] for p in nmps])
    nb2 = jnp.stack([p["b2"].reshape(1, 1) for p in nmps])
    fw1 = jnp.stack([p["W1"] for p in fmps])
    fb1 = jnp.stack([p["b1"].reshape(1, H) for p in fmps])
    fw2 = jnp.stack([p["W2"] for p in fmps])
    fb2 = jnp.stack([p["b2"].reshape(1, F) for p in fmps])
    return pl.pallas_call(
        _prep_kernel,
        grid=(_NB,),
        in_specs=[
            pl.BlockSpec((_BN, F), lambda i: (i, 0)),
            pl.BlockSpec((_BN, H), lambda i: (i, 0)),
            _full((NEXP, H, H)), _full((NEXP, 1, H)),
            _full((NEXP, H, 1)), _full((NEXP, 1, 1)),
            _full((NEXP, H, H)), _full((NEXP, 1, H)),
            _full((NEXP, H, F)), _full((NEXP, 1, F)),
        ],
        out_specs=[
            pl.BlockSpec((NEXP, _BN, F), lambda i: (0, i, 0)),
            pl.BlockSpec((_BN, NEXP), lambda i: (i, 0)),
        ],
        out_shape=[
            jax.ShapeDtypeStruct((NEXP, N, F), _f32),
            jax.ShapeDtypeStruct((N, NEXP), _f32),
        ],
    )(x, h, nw1, nb1, nw2, nb2, fw1, fb1, fw2, fb2)


def _edge_mlp_kernel(hs_ref, hd_ref, w1a, w1b, b1, w2, b2, o_ref):
    hs = hs_ref[...]
    hd = hd_ref[...]
    for e in range(NEXP):
        z = jnp.maximum(
            jnp.dot(hs, w1a[e], preferred_element_type=_f32)
            + jnp.dot(hd, w1b[e], preferred_element_type=_f32) + b1[e], 0.0)
        lg = jnp.dot(z, w2[e], preferred_element_type=_f32) + b2[e]
        o_ref[:, e:e + 1] = jax.nn.sigmoid(lg)


def _edge_mlp(hsrc, hdst, emps):
    w1a = jnp.stack([p["W1"][:H] for p in emps])
    w1b = jnp.stack([p["W1"][H:] for p in emps])
    b1 = jnp.stack([p["b1"].reshape(1, H) for p in emps])
    w2 = jnp.stack([p["W2"] for p in emps])
    b2 = jnp.stack([p["b2"].reshape(1, 1) for p in emps])
    return pl.pallas_call(
        _edge_mlp_kernel,
        grid=(E // _BE,),
        in_specs=[
            pl.BlockSpec((_BE, H), lambda i: (i, 0)),
            pl.BlockSpec((_BE, H), lambda i: (i, 0)),
            _full((NEXP, H, H)), _full((NEXP, H, H)), _full((NEXP, 1, H)),
            _full((NEXP, H, 1)), _full((NEXP, 1, 1)),
        ],
        out_specs=pl.BlockSpec((_BE, NEXP), lambda i: (i, 0)),
        out_shape=jax.ShapeDtypeStruct((E, NEXP), _f32),
    )(hsrc, hdst, w1a, w1b, b1, w2, b2)


def _pool_kernel(hc_ref, nm_ref, b_ref, cw_ref, cb_ref, o_ref, gacc, cacc):
    i = pl.program_id(0)

    @pl.when(i == 0)
    def _():
        gacc[...] = jnp.zeros((NEXP, NGRAPH, H), _f32)
        cacc[...] = jnp.zeros((NGRAPH, NEXP), _f32)

    bb = b_ref[0, 0]
    oh = (bb[None, :] == lax.broadcasted_iota(_i32, (NGRAPH, _BN), 0)
          ).astype(_f32)
    nm = nm_ref[...]
    cacc[...] += jnp.dot(oh, nm, preferred_element_type=_f32)
    for e in range(NEXP):
        gacc[e] += jnp.dot(oh, hc_ref[e] * nm[:, e:e + 1],
                           preferred_element_type=_f32)

    @pl.when(i == _NB - 1)
    def _():
        for e in range(NEXP):
            g = gacc[e] / jnp.maximum(cacc[...][:, e:e + 1], 1e-6)
            o_ref[e] = (jnp.dot(g, cw_ref[e], preferred_element_type=_f32)
                        + cb_ref[e])


def _pool_clf(hc, nm4, batch, clfps):
    cw = jnp.stack([p["W"] for p in clfps])
    cb = jnp.stack([p["b"].reshape(1, NCLS) for p in clfps])
    b3 = batch.reshape(_NB, 1, _BN).astype(_i32)
    return pl.pallas_call(
        _pool_kernel,
        grid=(_NB,),
        in_specs=[
            pl.BlockSpec((NEXP, _BN, H), lambda i: (0, i, 0)),
            pl.BlockSpec((_BN, NEXP), lambda i: (i, 0)),
            pl.BlockSpec((1, 1, _BN), lambda i: (i, 0, 0)),
            _full((NEXP, H, NCLS)), _full((NEXP, 1, NCLS)),
        ],
        out_specs=pl.BlockSpec((NEXP, NGRAPH, NCLS), lambda i: (0, 0, 0)),
        out_shape=jax.ShapeDtypeStruct((NEXP, NGRAPH, NCLS), _f32),
        scratch_shapes=[
            pltpu.VMEM((NEXP, NGRAPH, H), _f32),
            pltpu.VMEM((NGRAPH, NEXP), _f32),
        ],
    )(hc, nm4, b3, cw, cb)


_make_segsum = functools.lru_cache(None)(_make_segsum)
_make_gather_rows = functools.lru_cache(None)(_make_gather_rows)


def _segsum_w(h, src, dst, w):
    return _make_segsum(h.shape[1], True)(h, src, dst, w)


def _gather64(h, idx):
    return _make_gather_rows(H)(h, idx)


def _segsum_cols(h, src, dst, w):
    if h.shape[1] == H:
        return [_segsum_w(h, src, dst, w)]
    return [_segsum_w(h[:, :H], src, dst, w),
            _segsum_w(h[:, H:], src, dst, w)]


def kernel(x, edge_index, batch, params):
    src = edge_index[0]
    dst = edge_index[1]

    h = x
    ones_e = jnp.ones((E,), _f32)
    for p in params["causal"]:
        aggs = _segsum_cols(h, src, dst, ones_e)
        h = _gin_layer(h, aggs, p)

    xm4, nm4 = _prep(x, h, params["node_mask"], params["feat_mask"])

    hsrc = _gather64(h, src)
    hdst = _gather64(h, dst)
    em4 = _edge_mlp(hsrc, hdst, params["edge_mask"]).T

    hc = xm4
    for p in params["clf_enc"]:
        agg4s = []
        nhalf = hc.shape[2] // H
        for half in range(nhalf):
            part = hc[:, :, half * H:(half + 1) * H]
            agg4s.append(jnp.stack(
                [_segsum_w(part[e], src, dst, em4[e]) for e in range(NEXP)]))
        hc = _gin_layer4(hc, agg4s, p)

    return _pool_clf(hc, nm4, batch, params["clf"])


# speedup vs baseline: 1.8428x; 1.8428x over previous
import functools

import jax
import jax.numpy as jnp
from jax import lax
from jax.experimental import pallas as pl
from jax.experimental.pallas import tpu as pltpu
from jax.experimental.pallas import tpu_sc as plsc

N = 10000
E = 320000
F = 128
H = 64
NEXP = 4
NCLS = 10
NGRAPH = 128

NC, NS = 2, 16
EPC = E // NC
EPW = EPC // NS
SUB = 80
CHUNK = 400
NSUB = CHUNK // SUB
NCHUNK = EPW // CHUNK
SW = 624
ZR = 24
NZ = SW // ZR
TAIL = N - NS * SW

_f32 = jnp.float32
_i32 = jnp.int32


def _mesh():
    return plsc.VectorSubcoreMesh(core_axis_name="c", subcore_axis_name="s",
                                  num_cores=NC, num_subcores=NS)


def _maybe(cond, fn):
    if isinstance(cond, bool):
        if cond:
            fn()
    else:
        pl.when(cond)(fn)


def _zero_acc(zbuf, acc, sid, d):
    def zb(i, c):
        for j in range(d // 16):
            zbuf[i, pl.ds(j * 16, 16)] = jnp.zeros((16,), _f32)
        return c

    lax.fori_loop(0, ZR, zb, 0)

    def zc(i, c):
        pltpu.sync_copy(zbuf, acc.at[pl.ds(sid * SW + i * ZR, ZR)])
        return c

    lax.fori_loop(0, NZ, zc, 0)

    @pl.when(sid == NS - 1)
    def _():
        pltpu.sync_copy(zbuf.at[pl.ds(0, TAIL)], acc.at[pl.ds(NS * SW, TAIL)])


def _make_segsum(d, weighted):
    NSL = 3
    scratch = (
        [pltpu.VMEM((NSUB, SUB), _i32) for _ in range(NSL)]
        + [pltpu.VMEM((NSUB, SUB), _i32) for _ in range(NSL)]
        + [pltpu.VMEM((CHUNK,), _f32) for _ in range(NSL)]
        + [pltpu.VMEM((CHUNK, d), _f32) for _ in range(NSL)]
        + [pltpu.VMEM((ZR, d), _f32)]
        + [pltpu.VMEM_SHARED((N, d), _f32)]
        + [pltpu.SemaphoreType.DMA for _ in range(3 * NSL)]
    )

    def body(*refs):
        nin = 4 if weighted else 3
        h_hbm, src_hbm, dst_hbm = refs[0], refs[1], refs[2]
        w_hbm = refs[3] if weighted else None
        out_hbm = refs[nin]
        r = list(refs[nin + 1:])
        sidx = r[0:NSL]
        didx = r[NSL:2 * NSL]
        wbuf = r[2 * NSL:3 * NSL]
        rows = r[3 * NSL:4 * NSL]
        zbuf = r[4 * NSL]
        acc = r[4 * NSL + 1]
        lsem = r[4 * NSL + 2:4 * NSL + 2 + NSL]
        gsem = r[4 * NSL + 2 + NSL:4 * NSL + 2 + 2 * NSL]
        ssem = r[4 * NSL + 2 + 2 * NSL:]
        cid = lax.axis_index("c")
        sid = lax.axis_index("s")
        base = cid * EPC + sid * EPW

        _zero_acc(zbuf, acc, sid, d)
        plsc.subcore_barrier()

        def load_copies(g, s, f):
            off = base + g * CHUNK
            for j in range(NSUB):
                f(src_hbm.at[pl.ds(off + j * SUB, SUB)], sidx[s].at[j],
                  lsem[s])
                f(dst_hbm.at[pl.ds(off + j * SUB, SUB)], didx[s].at[j],
                  lsem[s])
            if weighted:
                f(w_hbm.at[pl.ds(off, CHUNK)], wbuf[s], lsem[s])

        def gather_copies(s, f):
            for j in range(NSUB):
                f(h_hbm.at[sidx[s].at[j]], rows[s].at[pl.ds(j * SUB, SUB)],
                  gsem[s])

        def scatter_copies(s, f):
            for j in range(NSUB):
                f(rows[s].at[pl.ds(j * SUB, SUB)], acc.at[didx[s].at[j]],
                  ssem[s], add=True)

        def fire(*a, **k):
            pltpu.async_copy(*a, **k)

        def drain(*a, **k):
            pltpu.make_async_copy(*a).wait()

        def mul(s):
            if not weighted:
                return

            @plsc.parallel_loop(0, CHUNK // 16, unroll=2)
            def _(t):
                wreg = wbuf[s][pl.ds(t * 16, 16)]
                bcs = [jnp.full((16,), wreg[k], _f32) for k in range(16)]
                for j in range(d // 16):
                    for k in range(16):
                        rows[s][t * 16 + k, pl.ds(j * 16, 16)] = (
                            rows[s][t * 16 + k, pl.ds(j * 16, 16)] * bcs[k])

        load_copies(0, 0, fire)
        load_copies(0, 0, drain)
        gather_copies(0, fire)

        def chunk_step(g, s):
            nxt = (s + 1) % NSL

            def pre():
                _maybe(g + 1 >= NSL, lambda: scatter_copies(nxt, drain))
                load_copies(g + 1, nxt, fire)

            _maybe(g + 1 < NCHUNK, pre)

            gather_copies(s, drain)
            mul(s)
            scatter_copies(s, fire)

            def post():
                load_copies(g + 1, nxt, drain)
                gather_copies(nxt, fire)

            _maybe(g + 1 < NCHUNK, post)

        def step(t, carry):
            for k in range(NSL):
                chunk_step(NSL * t + k, k)
            return carry

        lax.fori_loop(0, NCHUNK // NSL, step, 0)
        gl = NCHUNK - 1
        chunk_step(gl, gl % NSL)
        scatter_copies((gl - 2) % NSL, drain)
        scatter_copies((gl - 1) % NSL, drain)
        scatter_copies(gl % NSL, drain)

        plsc.subcore_barrier()
        pltpu.sync_copy(acc.at[pl.ds(sid * SW, SW)],
                        out_hbm.at[cid, pl.ds(sid * SW, SW)])

        @pl.when(sid == NS - 1)
        def _():
            pltpu.sync_copy(acc.at[pl.ds(NS * SW, TAIL)],
                            out_hbm.at[cid, pl.ds(NS * SW, TAIL)])

    return pl.kernel(
        body,
        out_type=jax.ShapeDtypeStruct((2, N, d), _f32),
        mesh=_mesh(),
        scratch_types=scratch,
        compiler_params=pltpu.CompilerParams(use_tc_tiling_on_sc=False),
    )


def _make_gather_rows(d):
    NSL = 3
    scratch = (
        [pltpu.VMEM((NSUB, SUB), _i32) for _ in range(NSL)]
        + [pltpu.VMEM((CHUNK, d), _f32) for _ in range(NSL)]
        + [pltpu.SemaphoreType.DMA for _ in range(3 * NSL)]
    )

    def body(h_hbm, idx_hbm, out_hbm, *r):
        sidx = r[0:NSL]
        rows = r[NSL:2 * NSL]
        lsem = r[2 * NSL:3 * NSL]
        gsem = r[3 * NSL:4 * NSL]
        osem = r[4 * NSL:5 * NSL]
        cid = lax.axis_index("c")
        sid = lax.axis_index("s")
        base = cid * EPC + sid * EPW

        def load_copies(g, s, f):
            off = base + g * CHUNK
            for j in range(NSUB):
                f(idx_hbm.at[pl.ds(off + j * SUB, SUB)], sidx[s].at[j],
                  lsem[s])

        def gather_copies(s, f):
            for j in range(NSUB):
                f(h_hbm.at[sidx[s].at[j]], rows[s].at[pl.ds(j * SUB, SUB)],
                  gsem[s])

        def out_copies(g, s, f):
            f(rows[s], out_hbm.at[pl.ds(base + g * CHUNK, CHUNK)], osem[s])

        def fire(*a, **k):
            pltpu.async_copy(*a, **k)

        def drain(*a, **k):
            pltpu.make_async_copy(*a, **k).wait()

        load_copies(0, 0, fire)
        load_copies(0, 0, drain)
        gather_copies(0, fire)

        def chunk_step(g, s):
            nxt = (s + 1) % NSL

            def pre():
                _maybe(g + 1 >= NSL, lambda: out_copies(g - 2, nxt, drain))
                load_copies(g + 1, nxt, fire)

            _maybe(g + 1 < NCHUNK, pre)

            gather_copies(s, drain)
            out_copies(g, s, fire)

            def post():
                load_copies(g + 1, nxt, drain)
                gather_copies(nxt, fire)

            _maybe(g + 1 < NCHUNK, post)

        def step(t, carry):
            for k in range(NSL):
                chunk_step(NSL * t + k, k)
            return carry

        lax.fori_loop(0, NCHUNK // NSL, step, 0)
        gl = NCHUNK - 1
        chunk_step(gl, gl % NSL)
        out_copies(gl - 2, (gl - 2) % NSL, drain)
        out_copies(gl - 1, (gl - 1) % NSL, drain)
        out_copies(gl, gl % NSL, drain)

    return pl.kernel(
        body,
        out_type=jax.ShapeDtypeStruct((E, d), _f32),
        mesh=_mesh(),
        scratch_types=scratch,
        compiler_params=pltpu.CompilerParams(use_tc_tiling_on_sc=False),
    )


_BN = 1000
_NB = N // _BN
_BE = 2000


def _full(shape):
    return pl.BlockSpec(shape, lambda *_: tuple(0 for _ in shape))


def _gin_mlp_kernel(nagg, h_ref, *refs):
    aggs = refs[:nagg]
    scale_ref, w1_ref, b1_ref, w2_ref, b2_ref, o_ref = refs[nagg:]
    hs = h_ref[...] * scale_ref[0, 0]
    if nagg == 1:
        a = hs + aggs[0][0] + aggs[0][1]
    else:
        a = jnp.concatenate(
            [hs[:, :H] + aggs[0][0] + aggs[0][1],
             hs[:, H:] + aggs[1][0] + aggs[1][1]], axis=1)
    t = jnp.maximum(
        jnp.dot(a, w1_ref[...], preferred_element_type=_f32) + b1_ref[...], 0.0)
    o = jnp.dot(t, w2_ref[...], preferred_element_type=_f32) + b2_ref[...]
    o_ref[...] = jnp.maximum(o, 0.0)


def _gin_layer(h, aggs, p):
    din = h.shape[1]
    scale = (1.0 + p["eps"]).reshape(1, 1).astype(_f32)
    return pl.pallas_call(
        functools.partial(_gin_mlp_kernel, len(aggs)),
        grid=(_NB,),
        in_specs=[
            pl.BlockSpec((_BN, din), lambda i: (i, 0)),
        ] + [
            pl.BlockSpec((2, _BN, H), lambda i: (0, i, 0)) for _ in aggs
        ] + [
            _full((1, 1)),
            _full((din, H)),
            _full((1, H)),
            _full((H, H)),
            _full((1, H)),
        ],
        out_specs=pl.BlockSpec((_BN, H), lambda i: (i, 0)),
        out_shape=jax.ShapeDtypeStruct((N, H), _f32),
    )(h, *aggs, scale, p["W1"], p["b1"].reshape(1, H), p["W2"],
      p["b2"].reshape(1, H))


def _gin_mlp4_kernel(nagg, h_ref, *refs):
    aggs = refs[:nagg]
    scale_ref, w1_ref, b1_ref, w2_ref, b2_ref, o_ref = refs[nagg:]
    hs = h_ref[0] * scale_ref[0, 0]
    if nagg == 1:
        a = hs + aggs[0][0, 0] + aggs[0][0, 1]
    else:
        a = jnp.concatenate(
            [hs[:, :H] + aggs[0][0, 0] + aggs[0][0, 1],
             hs[:, H:] + aggs[1][0, 0] + aggs[1][0, 1]], axis=1)
    t = jnp.maximum(
        jnp.dot(a, w1_ref[...], preferred_element_type=_f32) + b1_ref[...], 0.0)
    o = jnp.dot(t, w2_ref[...], preferred_element_type=_f32) + b2_ref[...]
    o_ref[0] = jnp.maximum(o, 0.0)


def _gin_layer4(hc, agg4s, p):
    din = hc.shape[2]
    scale = (1.0 + p["eps"]).reshape(1, 1).astype(_f32)
    return pl.pallas_call(
        functools.partial(_gin_mlp4_kernel, len(agg4s)),
        grid=(NEXP, _NB),
        in_specs=[
            pl.BlockSpec((1, _BN, din), lambda e, i: (e, i, 0)),
        ] + [
            pl.BlockSpec((1, 2, _BN, H), lambda e, i: (e, 0, i, 0))
            for _ in agg4s
        ] + [
            _full((1, 1)),
            _full((din, H)),
            _full((1, H)),
            _full((H, H)),
            _full((1, H)),
        ],
        out_specs=pl.BlockSpec((1, _BN, H), lambda e, i: (e, i, 0)),
        out_shape=jax.ShapeDtypeStruct((NEXP, N, H), _f32),
    )(hc, *agg4s, scale, p["W1"], p["b1"].reshape(1, H), p["W2"],
      p["b2"].reshape(1, H))


def _prep_kernel(x_ref, h_ref, nw1, nb1, nw2, nb2, fw1, fb1, fw2, fb2,
                 xm_ref, nm_ref):
    h = h_ref[...]
    x = x_ref[...]
    for e in range(NEXP):
        t = jnp.maximum(
            jnp.dot(h, nw1[e], preferred_element_type=_f32) + nb1[e], 0.0)
        nm = jax.nn.sigmoid(
            jnp.dot(t, nw2[e], preferred_element_type=_f32) + nb2[e])
        t2 = jnp.maximum(
            jnp.dot(h, fw1[e], preferred_element_type=_f32) + fb1[e], 0.0)
        fm = jax.nn.sigmoid(
            jnp.dot(t2, fw2[e], preferred_element_type=_f32) + fb2[e])
        xm_ref[e] = x * fm * nm
        nm_ref[:, e:e + 1] = nm


def _prep(x, h, nmps, fmps):
    nw1 = jnp.stack([p["W1"] for p in nmps])
    nb1 = jnp.stack([p["b1"].reshape(1, H) for p in nmps])
    nw2 = jnp.stack([p["W2"] for p in nmps])
    nb2 = jnp.stack([p["b2"].reshape(1, 1) for p in nmps])
    fw1 = jnp.stack([p["W1"] for p in fmps])
    fb1 = jnp.stack([p["b1"].reshape(1, H) for p in fmps])
    fw2 = jnp.stack([p["W2"] for p in fmps])
    fb2 = jnp.stack([p["b2"].reshape(1, F) for p in fmps])
    return pl.pallas_call(
        _prep_kernel,
        grid=(_NB,),
        in_specs=[
            pl.BlockSpec((_BN, F), lambda i: (i, 0)),
            pl.BlockSpec((_BN, H), lambda i: (i, 0)),
            _full((NEXP, H, H)), _full((NEXP, 1, H)),
            _full((NEXP, H, 1)), _full((NEXP, 1, 1)),
            _full((NEXP, H, H)), _full((NEXP, 1, H)),
            _full((NEXP, H, F)), _full((NEXP, 1, F)),
        ],
        out_specs=[
            pl.BlockSpec((NEXP, _BN, F), lambda i: (0, i, 0)),
            pl.BlockSpec((_BN, NEXP), lambda i: (i, 0)),
        ],
        out_shape=[
            jax.ShapeDtypeStruct((NEXP, N, F), _f32),
            jax.ShapeDtypeStruct((N, NEXP), _f32),
        ],
    )(x, h, nw1, nb1, nw2, nb2, fw1, fb1, fw2, fb2)


def _edge_mlp_kernel(hs_ref, hd_ref, w1a, w1b, b1, w2, b2, o_ref):
    hs = hs_ref[...]
    hd = hd_ref[...]
    for e in range(NEXP):
        z = jnp.maximum(
            jnp.dot(hs, w1a[e], preferred_element_type=_f32)
            + jnp.dot(hd, w1b[e], preferred_element_type=_f32) + b1[e], 0.0)
        lg = jnp.dot(z, w2[e], preferred_element_type=_f32) + b2[e]
        o_ref[:, e:e + 1] = jax.nn.sigmoid(lg)


def _edge_mlp(hsrc, hdst, emps):
    w1a = jnp.stack([p["W1"][:H] for p in emps])
    w1b = jnp.stack([p["W1"][H:] for p in emps])
    b1 = jnp.stack([p["b1"].reshape(1, H) for p in emps])
    w2 = jnp.stack([p["W2"] for p in emps])
    b2 = jnp.stack([p["b2"].reshape(1, 1) for p in emps])
    return pl.pallas_call(
        _edge_mlp_kernel,
        grid=(E // _BE,),
        in_specs=[
            pl.BlockSpec((_BE, H), lambda i: (i, 0)),
            pl.BlockSpec((_BE, H), lambda i: (i, 0)),
            _full((NEXP, H, H)), _full((NEXP, H, H)), _full((NEXP, 1, H)),
            _full((NEXP, H, 1)), _full((NEXP, 1, 1)),
        ],
        out_specs=pl.BlockSpec((_BE, NEXP), lambda i: (i, 0)),
        out_shape=jax.ShapeDtypeStruct((E, NEXP), _f32),
    )(hsrc, hdst, w1a, w1b, b1, w2, b2)


def _pool_kernel(hc_ref, nm_ref, b_ref, cw_ref, cb_ref, o_ref, gacc, cacc):
    i = pl.program_id(0)

    @pl.when(i == 0)
    def _():
        gacc[...] = jnp.zeros((NEXP, NGRAPH, H), _f32)
        cacc[...] = jnp.zeros((NGRAPH, NEXP), _f32)

    bb = b_ref[0, 0]
    oh = (bb[None, :] == lax.broadcasted_iota(_i32, (NGRAPH, _BN), 0)
          ).astype(_f32)
    nm = nm_ref[...]
    cacc[...] += jnp.dot(oh, nm, preferred_element_type=_f32)
    for e in range(NEXP):
        gacc[e] += jnp.dot(oh, hc_ref[e] * nm[:, e:e + 1],
                           preferred_element_type=_f32)

    @pl.when(i == _NB - 1)
    def _():
        for e in range(NEXP):
            g = gacc[e] / jnp.maximum(cacc[...][:, e:e + 1], 1e-6)
            o_ref[e] = (jnp.dot(g, cw_ref[e], preferred_element_type=_f32)
                        + cb_ref[e])


def _pool_clf(hc, nm4, batch, clfps):
    cw = jnp.stack([p["W"] for p in clfps])
    cb = jnp.stack([p["b"].reshape(1, NCLS) for p in clfps])
    b3 = batch.reshape(_NB, 1, _BN).astype(_i32)
    return pl.pallas_call(
        _pool_kernel,
        grid=(_NB,),
        in_specs=[
            pl.BlockSpec((NEXP, _BN, H), lambda i: (0, i, 0)),
            pl.BlockSpec((_BN, NEXP), lambda i: (i, 0)),
            pl.BlockSpec((1, 1, _BN), lambda i: (i, 0, 0)),
            _full((NEXP, H, NCLS)), _full((NEXP, 1, NCLS)),
        ],
        out_specs=pl.BlockSpec((NEXP, NGRAPH, NCLS), lambda i: (0, 0, 0)),
        out_shape=jax.ShapeDtypeStruct((NEXP, NGRAPH, NCLS), _f32),
        scratch_shapes=[
            pltpu.VMEM((NEXP, NGRAPH, H), _f32),
            pltpu.VMEM((NGRAPH, NEXP), _f32),
        ],
    )(hc, nm4, b3, cw, cb)


_make_segsum = functools.lru_cache(None)(_make_segsum)
_make_gather_rows = functools.lru_cache(None)(_make_gather_rows)


def _segsum_w(h, src, dst, w):
    return _make_segsum(h.shape[1], True)(h, src, dst, w)


def _segsum_unw(h, src, dst):
    return _make_segsum(h.shape[1], False)(h, src, dst)


def _gather64(h, idx):
    return _make_gather_rows(H)(h, idx)


def _segsum_cols(h, src, dst, w):
    def one(hh):
        if w is None:
            return _segsum_unw(hh, src, dst)
        return _segsum_w(hh, src, dst, w)

    if h.shape[1] == H:
        return [one(h)]
    return [one(h[:, :H]), one(h[:, H:])]


def kernel(x, edge_index, batch, params):
    src = edge_index[0]
    dst = edge_index[1]

    h = x
    for p in params["causal"]:
        aggs = _segsum_cols(h, src, dst, None)
        h = _gin_layer(h, aggs, p)

    xm4, nm4 = _prep(x, h, params["node_mask"], params["feat_mask"])

    hsrc = _gather64(h, src)
    hdst = _gather64(h, dst)
    em4 = _edge_mlp(hsrc, hdst, params["edge_mask"]).T

    hc = xm4
    for p in params["clf_enc"]:
        agg4s = []
        nhalf = hc.shape[2] // H
        for half in range(nhalf):
            part = hc[:, :, half * H:(half + 1) * H]
            agg4s.append(jnp.stack(
                [_segsum_w(part[e], src, dst, em4[e]) for e in range(NEXP)]))
        hc = _gin_layer4(hc, agg4s, p)

    return _pool_clf(hc, nm4, batch, params["clf"])


# speedup vs baseline: 1.9171x; 1.0403x over previous
import functools

import jax
import jax.numpy as jnp
from jax import lax
from jax.experimental import pallas as pl
from jax.experimental.pallas import tpu as pltpu
from jax.experimental.pallas import tpu_sc as plsc

N = 10000
E = 320000
F = 128
H = 64
NEXP = 4
NCLS = 10
NGRAPH = 128

NC, NS = 2, 16
EPC = E // NC
EPW = EPC // NS
SUB = 80
CHUNK = 400
NSUB = CHUNK // SUB
NCHUNK = EPW // CHUNK
SW = 624
ZR = 24
NZ = SW // ZR
TAIL = N - NS * SW

_f32 = jnp.float32
_i32 = jnp.int32


def _mesh():
    return plsc.VectorSubcoreMesh(core_axis_name="c", subcore_axis_name="s",
                                  num_cores=NC, num_subcores=NS)


def _maybe(cond, fn):
    if isinstance(cond, bool):
        if cond:
            fn()
    else:
        pl.when(cond)(fn)


def _zero_acc(zbuf, acc, sid, d):
    def zb(i, c):
        for j in range(d // 16):
            zbuf[i, pl.ds(j * 16, 16)] = jnp.zeros((16,), _f32)
        return c

    lax.fori_loop(0, ZR, zb, 0)

    def zc(i, c):
        pltpu.sync_copy(zbuf, acc.at[pl.ds(sid * SW + i * ZR, ZR)])
        return c

    lax.fori_loop(0, NZ, zc, 0)

    @pl.when(sid == NS - 1)
    def _():
        pltpu.sync_copy(zbuf.at[pl.ds(0, TAIL)], acc.at[pl.ds(NS * SW, TAIL)])


def _make_segsum(d, weighted, npass=1):
    NSL = 3
    scratch = (
        [pltpu.VMEM((NSUB, SUB), _i32) for _ in range(NSL)]
        + [pltpu.VMEM((NSUB, SUB), _i32) for _ in range(NSL)]
        + [pltpu.VMEM((CHUNK,), _f32) for _ in range(NSL)]
        + [pltpu.VMEM((CHUNK, d), _f32) for _ in range(NSL)]
        + [pltpu.VMEM((ZR, d), _f32)]
        + [pltpu.VMEM_SHARED((N, d), _f32)]
        + [pltpu.SemaphoreType.DMA for _ in range(3 * NSL)]
    )

    def body(*refs):
        nin = 4 if weighted else 3
        h_hbm, src_hbm, dst_hbm = refs[0], refs[1], refs[2]
        w_hbm = refs[3] if weighted else None
        out_hbm = refs[nin]
        r = list(refs[nin + 1:])
        sidx = r[0:NSL]
        didx = r[NSL:2 * NSL]
        wbuf = r[2 * NSL:3 * NSL]
        rows = r[3 * NSL:4 * NSL]
        zbuf = r[4 * NSL]
        acc = r[4 * NSL + 1]
        lsem = r[4 * NSL + 2:4 * NSL + 2 + NSL]
        gsem = r[4 * NSL + 2 + NSL:4 * NSL + 2 + 2 * NSL]
        ssem = r[4 * NSL + 2 + 2 * NSL:]
        cid = lax.axis_index("c")
        sid = lax.axis_index("s")
        base = cid * EPC + sid * EPW

        def pass_body(p, carry):
            h_p = h_hbm.at[p] if npass > 1 else h_hbm
            w_p = w_hbm.at[p] if (weighted and npass > 1) else w_hbm
            out_p = out_hbm.at[p] if npass > 1 else out_hbm

            _zero_acc(zbuf, acc, sid, d)
            plsc.subcore_barrier()

            def load_copies(g, s, f):
                off = base + g * CHUNK
                for j in range(NSUB):
                    f(src_hbm.at[pl.ds(off + j * SUB, SUB)], sidx[s].at[j],
                      lsem[s])
                    f(dst_hbm.at[pl.ds(off + j * SUB, SUB)], didx[s].at[j],
                      lsem[s])
                if weighted:
                    f(w_p.at[pl.ds(off, CHUNK)], wbuf[s], lsem[s])

            def gather_copies(s, f):
                for j in range(NSUB):
                    f(h_p.at[sidx[s].at[j]], rows[s].at[pl.ds(j * SUB, SUB)],
                      gsem[s])

            def scatter_copies(s, f):
                for j in range(NSUB):
                    f(rows[s].at[pl.ds(j * SUB, SUB)], acc.at[didx[s].at[j]],
                      ssem[s], add=True)

            def fire(*a, **k):
                pltpu.async_copy(*a, **k)

            def drain(*a, **k):
                pltpu.make_async_copy(*a).wait()

            def mul(s):
                if not weighted:
                    return

                @plsc.parallel_loop(0, CHUNK // 16, unroll=2)
                def _(t):
                    wreg = wbuf[s][pl.ds(t * 16, 16)]
                    bcs = [jnp.full((16,), wreg[k], _f32) for k in range(16)]
                    for j in range(d // 16):
                        for k in range(16):
                            rows[s][t * 16 + k, pl.ds(j * 16, 16)] = (
                                rows[s][t * 16 + k, pl.ds(j * 16, 16)]
                                * bcs[k])

            load_copies(0, 0, fire)
            load_copies(0, 0, drain)
            gather_copies(0, fire)

            def chunk_step(g, s):
                nxt = (s + 1) % NSL

                def pre():
                    _maybe(g + 1 >= NSL, lambda: scatter_copies(nxt, drain))
                    load_copies(g + 1, nxt, fire)

                _maybe(g + 1 < NCHUNK, pre)

                gather_copies(s, drain)
                mul(s)
                scatter_copies(s, fire)

                def post():
                    load_copies(g + 1, nxt, drain)
                    gather_copies(nxt, fire)

                _maybe(g + 1 < NCHUNK, post)

            def step(t, carry2):
                for k in range(NSL):
                    chunk_step(NSL * t + k, k)
                return carry2

            lax.fori_loop(0, NCHUNK // NSL, step, 0)
            gl = NCHUNK - 1
            chunk_step(gl, gl % NSL)
            scatter_copies((gl - 2) % NSL, drain)
            scatter_copies((gl - 1) % NSL, drain)
            scatter_copies(gl % NSL, drain)

            plsc.subcore_barrier()
            pltpu.sync_copy(acc.at[pl.ds(sid * SW, SW)],
                            out_p.at[cid, pl.ds(sid * SW, SW)])

            @pl.when(sid == NS - 1)
            def _():
                pltpu.sync_copy(acc.at[pl.ds(NS * SW, TAIL)],
                                out_p.at[cid, pl.ds(NS * SW, TAIL)])

            plsc.subcore_barrier()
            return carry

        if npass > 1:
            lax.fori_loop(0, npass, pass_body, 0)
        else:
            pass_body(0, 0)

    oshape = (npass, 2, N, d) if npass > 1 else (2, N, d)
    return pl.kernel(
        body,
        out_type=jax.ShapeDtypeStruct(oshape, _f32),
        mesh=_mesh(),
        scratch_types=scratch,
        compiler_params=pltpu.CompilerParams(use_tc_tiling_on_sc=False),
    )


def _make_gather_rows(d):
    NSL = 3
    scratch = (
        [pltpu.VMEM((NSUB, SUB), _i32) for _ in range(NSL)]
        + [pltpu.VMEM((CHUNK, d), _f32) for _ in range(NSL)]
        + [pltpu.SemaphoreType.DMA for _ in range(3 * NSL)]
    )

    def body(h_hbm, idx_hbm, out_hbm, *r):
        sidx = r[0:NSL]
        rows = r[NSL:2 * NSL]
        lsem = r[2 * NSL:3 * NSL]
        gsem = r[3 * NSL:4 * NSL]
        osem = r[4 * NSL:5 * NSL]
        cid = lax.axis_index("c")
        sid = lax.axis_index("s")
        base = cid * EPC + sid * EPW

        def pass_body(p, carry):
            idx_p = idx_hbm.at[p]
            out_p = out_hbm.at[p]

            def load_copies(g, s, f):
                off = base + g * CHUNK
                for j in range(NSUB):
                    f(idx_p.at[pl.ds(off + j * SUB, SUB)], sidx[s].at[j],
                      lsem[s])

            def gather_copies(s, f):
                for j in range(NSUB):
                    f(h_hbm.at[sidx[s].at[j]],
                      rows[s].at[pl.ds(j * SUB, SUB)], gsem[s])

            def out_copies(g, s, f):
                f(rows[s], out_p.at[pl.ds(base + g * CHUNK, CHUNK)], osem[s])

            def fire(*a, **k):
                pltpu.async_copy(*a, **k)

            def drain(*a, **k):
                pltpu.make_async_copy(*a).wait()

            load_copies(0, 0, fire)
            load_copies(0, 0, drain)
            gather_copies(0, fire)

            def chunk_step(g, s):
                nxt = (s + 1) % NSL

                def pre():
                    _maybe(g + 1 >= NSL,
                           lambda: out_copies(g - 2, nxt, drain))
                    load_copies(g + 1, nxt, fire)

                _maybe(g + 1 < NCHUNK, pre)

                gather_copies(s, drain)
                out_copies(g, s, fire)

                def post():
                    load_copies(g + 1, nxt, drain)
                    gather_copies(nxt, fire)

                _maybe(g + 1 < NCHUNK, post)

            def step(t, carry2):
                for k in range(NSL):
                    chunk_step(NSL * t + k, k)
                return carry2

            lax.fori_loop(0, NCHUNK // NSL, step, 0)
            gl = NCHUNK - 1
            chunk_step(gl, gl % NSL)
            out_copies(gl - 2, (gl - 2) % NSL, drain)
            out_copies(gl - 1, (gl - 1) % NSL, drain)
            out_copies(gl, gl % NSL, drain)
            return carry

        lax.fori_loop(0, 2, pass_body, 0)

    return pl.kernel(
        body,
        out_type=jax.ShapeDtypeStruct((2, E, d), _f32),
        mesh=_mesh(),
        scratch_types=scratch,
        compiler_params=pltpu.CompilerParams(use_tc_tiling_on_sc=False),
    )


_BN = 1000
_NB = N // _BN
_BE = 2000


def _full(shape):
    return pl.BlockSpec(shape, lambda *_: tuple(0 for _ in shape))


def _gin_mlp_kernel(nagg, h_ref, *refs):
    aggs = refs[:nagg]
    scale_ref, w1_ref, b1_ref, w2_ref, b2_ref, o_ref = refs[nagg:]
    hs = h_ref[...] * scale_ref[0, 0]
    if nagg == 1:
        a = hs + aggs[0][0] + aggs[0][1]
    else:
        a = jnp.concatenate(
            [hs[:, :H] + aggs[0][0] + aggs[0][1],
             hs[:, H:] + aggs[1][0] + aggs[1][1]], axis=1)
    t = jnp.maximum(
        jnp.dot(a, w1_ref[...], preferred_element_type=_f32) + b1_ref[...], 0.0)
    o = jnp.dot(t, w2_ref[...], preferred_element_type=_f32) + b2_ref[...]
    o_ref[...] = jnp.maximum(o, 0.0)


def _gin_layer(h, aggs, p):
    din = h.shape[1]
    scale = (1.0 + p["eps"]).reshape(1, 1).astype(_f32)
    return pl.pallas_call(
        functools.partial(_gin_mlp_kernel, len(aggs)),
        grid=(_NB,),
        in_specs=[
            pl.BlockSpec((_BN, din), lambda i: (i, 0)),
        ] + [
            pl.BlockSpec((2, _BN, H), lambda i: (0, i, 0)) for _ in aggs
        ] + [
            _full((1, 1)),
            _full((din, H)),
            _full((1, H)),
            _full((H, H)),
            _full((1, H)),
        ],
        out_specs=pl.BlockSpec((_BN, H), lambda i: (i, 0)),
        out_shape=jax.ShapeDtypeStruct((N, H), _f32),
    )(h, *aggs, scale, p["W1"], p["b1"].reshape(1, H), p["W2"],
      p["b2"].reshape(1, H))


def _gin_mlp4_kernel(nagg, h_ref, *refs):
    aggs = refs[:nagg]
    scale_ref, w1_ref, b1_ref, w2_ref, b2_ref, o_ref = refs[nagg:]
    hs = h_ref[0] * scale_ref[0, 0]
    if nagg == 1:
        a = hs + aggs[0][0, 0] + aggs[0][0, 1]
    else:
        a = jnp.concatenate(
            [hs[:, :H] + aggs[0][0, 0] + aggs[0][0, 1],
             hs[:, H:] + aggs[1][0, 0] + aggs[1][0, 1]], axis=1)
    t = jnp.maximum(
        jnp.dot(a, w1_ref[...], preferred_element_type=_f32) + b1_ref[...], 0.0)
    o = jnp.dot(t, w2_ref[...], preferred_element_type=_f32) + b2_ref[...]
    o_ref[0] = jnp.maximum(o, 0.0)


def _gin_layer4(hc, agg4s, p):
    din = hc.shape[2]
    scale = (1.0 + p["eps"]).reshape(1, 1).astype(_f32)
    return pl.pallas_call(
        functools.partial(_gin_mlp4_kernel, len(agg4s)),
        grid=(NEXP, _NB),
        in_specs=[
            pl.BlockSpec((1, _BN, din), lambda e, i: (e, i, 0)),
        ] + [
            pl.BlockSpec((1, 2, _BN, H), lambda e, i: (e, 0, i, 0))
            for _ in agg4s
        ] + [
            _full((1, 1)),
            _full((din, H)),
            _full((1, H)),
            _full((H, H)),
            _full((1, H)),
        ],
        out_specs=pl.BlockSpec((1, _BN, H), lambda e, i: (e, i, 0)),
        out_shape=jax.ShapeDtypeStruct((NEXP, N, H), _f32),
    )(hc, *agg4s, scale, p["W1"], p["b1"].reshape(1, H), p["W2"],
      p["b2"].reshape(1, H))


def _prep_kernel(x_ref, h_ref, nw1, nb1, nw2, nb2, fw1, fb1, fw2, fb2,
                 xm_ref, nm_ref):
    h = h_ref[...]
    x = x_ref[...]
    for e in range(NEXP):
        t = jnp.maximum(
            jnp.dot(h, nw1[e], preferred_element_type=_f32) + nb1[e], 0.0)
        nm = jax.nn.sigmoid(
            jnp.dot(t, nw2[e], preferred_element_type=_f32) + nb2[e])
        t2 = jnp.maximum(
            jnp.dot(h, fw1[e], preferred_element_type=_f32) + fb1[e], 0.0)
        fm = jax.nn.sigmoid(
            jnp.dot(t2, fw2[e], preferred_element_type=_f32) + fb2[e])
        xm_ref[e] = x * fm * nm
        nm_ref[:, e:e + 1] = nm


def _prep(x, h, nmps, fmps):
    nw1 = jnp.stack([p["W1"] for p in nmps])
    nb1 = jnp.stack([p["b1"].reshape(1, H) for p in nmps])
    nw2 = jnp.stack([p["W2"] for p in nmps])
    nb2 = jnp.stack([p["b2"].reshape(1, 1) for p in nmps])
    fw1 = jnp.stack([p["W1"] for p in fmps])
    fb1 = jnp.stack([p["b1"].reshape(1, H) for p in fmps])
    fw2 = jnp.stack([p["W2"] for p in fmps])
    fb2 = jnp.stack([p["b2"].reshape(1, F) for p in fmps])
    return pl.pallas_call(
        _prep_kernel,
        grid=(_NB,),
        in_specs=[
            pl.BlockSpec((_BN, F), lambda i: (i, 0)),
            pl.BlockSpec((_BN, H), lambda i: (i, 0)),
            _full((NEXP, H, H)), _full((NEXP, 1, H)),
            _full((NEXP, H, 1)), _full((NEXP, 1, 1)),
            _full((NEXP, H, H)), _full((NEXP, 1, H)),
            _full((NEXP, H, F)), _full((NEXP, 1, F)),
        ],
        out_specs=[
            pl.BlockSpec((NEXP, _BN, F), lambda i: (0, i, 0)),
            pl.BlockSpec((_BN, NEXP), lambda i: (i, 0)),
        ],
        out_shape=[
            jax.ShapeDtypeStruct((NEXP, N, F), _f32),
            jax.ShapeDtypeStruct((N, NEXP), _f32),
        ],
    )(x, h, nw1, nb1, nw2, nb2, fw1, fb1, fw2, fb2)


def _edge_mlp_kernel(hs_ref, hd_ref, w1a, w1b, b1, w2, b2, o_ref):
    hs = hs_ref[...]
    hd = hd_ref[...]
    for e in range(NEXP):
        z = jnp.maximum(
            jnp.dot(hs, w1a[e], preferred_element_type=_f32)
            + jnp.dot(hd, w1b[e], preferred_element_type=_f32) + b1[e], 0.0)
        lg = jnp.dot(z, w2[e], preferred_element_type=_f32) + b2[e]
        o_ref[:, e:e + 1] = jax.nn.sigmoid(lg)


def _edge_mlp(hsrc, hdst, emps):
    w1a = jnp.stack([p["W1"][:H] for p in emps])
    w1b = jnp.stack([p["W1"][H:] for p in emps])
    b1 = jnp.stack([p["b1"].reshape(1, H) for p in emps])
    w2 = jnp.stack([p["W2"] for p in emps])
    b2 = jnp.stack([p["b2"].reshape(1, 1) for p in emps])
    return pl.pallas_call(
        _edge_mlp_kernel,
        grid=(E // _BE,),
        in_specs=[
            pl.BlockSpec((_BE, H), lambda i: (i, 0)),
            pl.BlockSpec((_BE, H), lambda i: (i, 0)),
            _full((NEXP, H, H)), _full((NEXP, H, H)), _full((NEXP, 1, H)),
            _full((NEXP, H, 1)), _full((NEXP, 1, 1)),
        ],
        out_specs=pl.BlockSpec((_BE, NEXP), lambda i: (i, 0)),
        out_shape=jax.ShapeDtypeStruct((E, NEXP), _f32),
    )(hsrc, hdst, w1a, w1b, b1, w2, b2)


def _pool_kernel(hc_ref, nm_ref, b_ref, cw_ref, cb_ref, o_ref, gacc, cacc):
    i = pl.program_id(0)

    @pl.when(i == 0)
    def _():
        gacc[...] = jnp.zeros((NEXP, NGRAPH, H), _f32)
        cacc[...] = jnp.zeros((NGRAPH, NEXP), _f32)

    bb = b_ref[0, 0]
    oh = (bb[None, :] == lax.broadcasted_iota(_i32, (NGRAPH, _BN), 0)
          ).astype(_f32)
    nm = nm_ref[...]
    cacc[...] += jnp.dot(oh, nm, preferred_element_type=_f32)
    for e in range(NEXP):
        gacc[e] += jnp.dot(oh, hc_ref[e] * nm[:, e:e + 1],
                           preferred_element_type=_f32)

    @pl.when(i == _NB - 1)
    def _():
        for e in range(NEXP):
            g = gacc[e] / jnp.maximum(cacc[...][:, e:e + 1], 1e-6)
            o_ref[e] = (jnp.dot(g, cw_ref[e], preferred_element_type=_f32)
                        + cb_ref[e])


def _pool_clf(hc, nm4, batch, clfps):
    cw = jnp.stack([p["W"] for p in clfps])
    cb = jnp.stack([p["b"].reshape(1, NCLS) for p in clfps])
    b3 = batch.reshape(_NB, 1, _BN).astype(_i32)
    return pl.pallas_call(
        _pool_kernel,
        grid=(_NB,),
        in_specs=[
            pl.BlockSpec((NEXP, _BN, H), lambda i: (0, i, 0)),
            pl.BlockSpec((_BN, NEXP), lambda i: (i, 0)),
            pl.BlockSpec((1, 1, _BN), lambda i: (i, 0, 0)),
            _full((NEXP, H, NCLS)), _full((NEXP, 1, NCLS)),
        ],
        out_specs=pl.BlockSpec((NEXP, NGRAPH, NCLS), lambda i: (0, 0, 0)),
        out_shape=jax.ShapeDtypeStruct((NEXP, NGRAPH, NCLS), _f32),
        scratch_shapes=[
            pltpu.VMEM((NEXP, NGRAPH, H), _f32),
            pltpu.VMEM((NGRAPH, NEXP), _f32),
        ],
    )(hc, nm4, b3, cw, cb)


_make_segsum = functools.lru_cache(None)(_make_segsum)
_make_gather_rows = functools.lru_cache(None)(_make_gather_rows)


def _segsum_w4(h4, src, dst, w4):
    return _make_segsum(h4.shape[2], True, npass=NEXP)(h4, src, dst, w4)


def _segsum_unw(h, src, dst):
    return _make_segsum(h.shape[1], False)(h, src, dst)


def _gather64(h, edge_index):
    return _make_gather_rows(H)(h, edge_index)


def _segsum_cols(h, src, dst, w):
    def one(hh):
        if w is None:
            return _segsum_unw(hh, src, dst)
        return _segsum_w(hh, src, dst, w)

    if h.shape[1] == H:
        return [one(h)]
    return [one(h[:, :H]), one(h[:, H:])]


def kernel(x, edge_index, batch, params):
    src = edge_index[0]
    dst = edge_index[1]

    h = x
    for p in params["causal"]:
        aggs = _segsum_cols(h, src, dst, None)
        h = _gin_layer(h, aggs, p)

    xm4, nm4 = _prep(x, h, params["node_mask"], params["feat_mask"])

    hsd = _gather64(h, edge_index)
    em4 = _edge_mlp(hsd[0], hsd[1], params["edge_mask"]).T

    hc = xm4
    for p in params["clf_enc"]:
        agg4s = []
        nhalf = hc.shape[2] // H
        for half in range(nhalf):
            part = hc[:, :, half * H:(half + 1) * H]
            agg4s.append(_segsum_w4(part, src, dst, em4))
        hc = _gin_layer4(hc, agg4s, p)

    return _pool_clf(hc, nm4, batch, params["clf"])


# speedup vs baseline: 2.0730x; 1.0813x over previous
import functools

import jax
import jax.numpy as jnp
from jax import lax
from jax.experimental import pallas as pl
from jax.experimental.pallas import tpu as pltpu
from jax.experimental.pallas import tpu_sc as plsc

N = 10000
E = 320000
F = 128
H = 64
NEXP = 4
NCLS = 10
NGRAPH = 128

NC, NS = 2, 16
EPC = E // NC
EPW = EPC // NS
SUB = 80
CHUNK = 400
NSUB = CHUNK // SUB
NCHUNK = EPW // CHUNK
SW = 624
ZR = 104
NZ = SW // ZR
TAIL = N - NS * SW

_f32 = jnp.float32
_i32 = jnp.int32


def _mesh():
    return plsc.VectorSubcoreMesh(core_axis_name="c", subcore_axis_name="s",
                                  num_cores=NC, num_subcores=NS)


def _maybe(cond, fn):
    if isinstance(cond, bool):
        if cond:
            fn()
    else:
        pl.when(cond)(fn)


def _zero_acc(zbuf, acc, sid, d, zsem):
    @plsc.parallel_loop(0, ZR)
    def _(i):
        for j in range(d // 16):
            zbuf[i, pl.ds(j * 16, 16)] = jnp.zeros((16,), _f32)

    for i in range(NZ):
        pltpu.async_copy(zbuf, acc.at[pl.ds(sid * SW + i * ZR, ZR)], zsem)

    @pl.when(sid == NS - 1)
    def _():
        pltpu.async_copy(zbuf.at[pl.ds(0, TAIL)],
                         acc.at[pl.ds(NS * SW, TAIL)], zsem)

    for i in range(NZ):
        pltpu.make_async_copy(
            zbuf, acc.at[pl.ds(sid * SW + i * ZR, ZR)], zsem).wait()

    @pl.when(sid == NS - 1)
    def _():
        pltpu.make_async_copy(zbuf.at[pl.ds(0, TAIL)],
                              acc.at[pl.ds(NS * SW, TAIL)], zsem).wait()


def _make_segsum(d, weighted, npass=1):
    NSL = 3
    scratch = (
        [pltpu.VMEM((NSUB, SUB), _i32) for _ in range(NSL)]
        + [pltpu.VMEM((NSUB, SUB), _i32) for _ in range(NSL)]
        + [pltpu.VMEM((CHUNK,), _f32) for _ in range(NSL)]
        + [pltpu.VMEM((CHUNK, d), _f32) for _ in range(NSL)]
        + [pltpu.VMEM((ZR, d), _f32)]
        + [pltpu.VMEM_SHARED((N, d), _f32)]
        + [pltpu.SemaphoreType.DMA for _ in range(3 * NSL + 1)]
    )

    def body(*refs):
        nin = 4 if weighted else 3
        h_hbm, src_hbm, dst_hbm = refs[0], refs[1], refs[2]
        w_hbm = refs[3] if weighted else None
        out_hbm = refs[nin]
        r = list(refs[nin + 1:])
        sidx = r[0:NSL]
        didx = r[NSL:2 * NSL]
        wbuf = r[2 * NSL:3 * NSL]
        rows = r[3 * NSL:4 * NSL]
        zbuf = r[4 * NSL]
        acc = r[4 * NSL + 1]
        lsem = r[4 * NSL + 2:4 * NSL + 2 + NSL]
        gsem = r[4 * NSL + 2 + NSL:4 * NSL + 2 + 2 * NSL]
        ssem = r[4 * NSL + 2 + 2 * NSL:4 * NSL + 2 + 3 * NSL]
        zsem = r[4 * NSL + 2 + 3 * NSL]
        cid = lax.axis_index("c")
        sid = lax.axis_index("s")
        base = cid * EPC + sid * EPW

        def pass_body(p, carry):
            h_p = h_hbm.at[p] if npass > 1 else h_hbm
            w_p = w_hbm.at[p] if (weighted and npass > 1) else w_hbm
            out_p = out_hbm.at[p] if npass > 1 else out_hbm

            _zero_acc(zbuf, acc, sid, d, zsem)
            plsc.subcore_barrier()

            def load_copies(g, s, f):
                off = base + g * CHUNK
                for j in range(NSUB):
                    f(src_hbm.at[pl.ds(off + j * SUB, SUB)], sidx[s].at[j],
                      lsem[s])
                    f(dst_hbm.at[pl.ds(off + j * SUB, SUB)], didx[s].at[j],
                      lsem[s])
                if weighted:
                    f(w_p.at[pl.ds(off, CHUNK)], wbuf[s], lsem[s])

            def gather_copies(s, f):
                for j in range(NSUB):
                    f(h_p.at[sidx[s].at[j]], rows[s].at[pl.ds(j * SUB, SUB)],
                      gsem[s])

            def scatter_copies(s, f):
                for j in range(NSUB):
                    f(rows[s].at[pl.ds(j * SUB, SUB)], acc.at[didx[s].at[j]],
                      ssem[s], add=True)

            def fire(*a, **k):
                pltpu.async_copy(*a, **k)

            def drain(*a, **k):
                pltpu.make_async_copy(*a).wait()

            def mul(s):
                if not weighted:
                    return

                @plsc.parallel_loop(0, CHUNK // 16, unroll=2)
                def _(t):
                    wreg = wbuf[s][pl.ds(t * 16, 16)]
                    bcs = [jnp.full((16,), wreg[k], _f32) for k in range(16)]
                    for j in range(d // 16):
                        for k in range(16):
                            rows[s][t * 16 + k, pl.ds(j * 16, 16)] = (
                                rows[s][t * 16 + k, pl.ds(j * 16, 16)]
                                * bcs[k])

            load_copies(0, 0, fire)
            load_copies(0, 0, drain)
            gather_copies(0, fire)

            def chunk_step(g, s):
                nxt = (s + 1) % NSL

                def pre():
                    _maybe(g + 1 >= NSL, lambda: scatter_copies(nxt, drain))
                    load_copies(g + 1, nxt, fire)

                _maybe(g + 1 < NCHUNK, pre)

                gather_copies(s, drain)
                mul(s)
                scatter_copies(s, fire)

                def post():
                    load_copies(g + 1, nxt, drain)
                    gather_copies(nxt, fire)

                _maybe(g + 1 < NCHUNK, post)

            def step(t, carry2):
                for k in range(NSL):
                    chunk_step(NSL * t + k, k)
                return carry2

            lax.fori_loop(0, NCHUNK // NSL, step, 0)
            gl = NCHUNK - 1
            chunk_step(gl, gl % NSL)
            scatter_copies((gl - 2) % NSL, drain)
            scatter_copies((gl - 1) % NSL, drain)
            scatter_copies(gl % NSL, drain)

            plsc.subcore_barrier()
            pltpu.sync_copy(acc.at[pl.ds(sid * SW, SW)],
                            out_p.at[cid, pl.ds(sid * SW, SW)])

            @pl.when(sid == NS - 1)
            def _():
                pltpu.sync_copy(acc.at[pl.ds(NS * SW, TAIL)],
                                out_p.at[cid, pl.ds(NS * SW, TAIL)])

            plsc.subcore_barrier()
            return carry

        if npass > 1:
            lax.fori_loop(0, npass, pass_body, 0)
        else:
            pass_body(0, 0)

    oshape = (npass, 2, N, d) if npass > 1 else (2, N, d)
    return pl.kernel(
        body,
        out_type=jax.ShapeDtypeStruct(oshape, _f32),
        mesh=_mesh(),
        scratch_types=scratch,
        compiler_params=pltpu.CompilerParams(use_tc_tiling_on_sc=False),
    )


def _make_gather_rows(d):
    NSL = 3
    scratch = (
        [pltpu.VMEM((NSUB, SUB), _i32) for _ in range(NSL)]
        + [pltpu.VMEM((CHUNK, d), _f32) for _ in range(NSL)]
        + [pltpu.SemaphoreType.DMA for _ in range(3 * NSL)]
    )

    def body(h_hbm, idx_hbm, out_hbm, *r):
        sidx = r[0:NSL]
        rows = r[NSL:2 * NSL]
        lsem = r[2 * NSL:3 * NSL]
        gsem = r[3 * NSL:4 * NSL]
        osem = r[4 * NSL:5 * NSL]
        cid = lax.axis_index("c")
        sid = lax.axis_index("s")
        base = cid * EPC + sid * EPW

        def pass_body(p, carry):
            idx_p = idx_hbm.at[p]
            out_p = out_hbm.at[p]

            def load_copies(g, s, f):
                off = base + g * CHUNK
                for j in range(NSUB):
                    f(idx_p.at[pl.ds(off + j * SUB, SUB)], sidx[s].at[j],
                      lsem[s])

            def gather_copies(s, f):
                for j in range(NSUB):
                    f(h_hbm.at[sidx[s].at[j]],
                      rows[s].at[pl.ds(j * SUB, SUB)], gsem[s])

            def out_copies(g, s, f):
                f(rows[s], out_p.at[pl.ds(base + g * CHUNK, CHUNK)], osem[s])

            def fire(*a, **k):
                pltpu.async_copy(*a, **k)

            def drain(*a, **k):
                pltpu.make_async_copy(*a).wait()

            load_copies(0, 0, fire)
            load_copies(0, 0, drain)
            gather_copies(0, fire)

            def chunk_step(g, s):
                nxt = (s + 1) % NSL

                def pre():
                    _maybe(g + 1 >= NSL,
                           lambda: out_copies(g - 2, nxt, drain))
                    load_copies(g + 1, nxt, fire)

                _maybe(g + 1 < NCHUNK, pre)

                gather_copies(s, drain)
                out_copies(g, s, fire)

                def post():
                    load_copies(g + 1, nxt, drain)
                    gather_copies(nxt, fire)

                _maybe(g + 1 < NCHUNK, post)

            def step(t, carry2):
                for k in range(NSL):
                    chunk_step(NSL * t + k, k)
                return carry2

            lax.fori_loop(0, NCHUNK // NSL, step, 0)
            gl = NCHUNK - 1
            chunk_step(gl, gl % NSL)
            out_copies(gl - 2, (gl - 2) % NSL, drain)
            out_copies(gl - 1, (gl - 1) % NSL, drain)
            out_copies(gl, gl % NSL, drain)
            return carry

        lax.fori_loop(0, 2, pass_body, 0)

    return pl.kernel(
        body,
        out_type=jax.ShapeDtypeStruct((2, E, d), _f32),
        mesh=_mesh(),
        scratch_types=scratch,
        compiler_params=pltpu.CompilerParams(use_tc_tiling_on_sc=False),
    )


_BN = 1000
_NB = N // _BN
_BE = 2000


def _full(shape):
    return pl.BlockSpec(shape, lambda *_: tuple(0 for _ in shape))


def _gin_mlp_kernel(nagg, h_ref, *refs):
    aggs = refs[:nagg]
    scale_ref, w1_ref, b1_ref, w2_ref, b2_ref, o_ref = refs[nagg:]
    hs = h_ref[...] * scale_ref[0, 0]
    if nagg == 1:
        a = hs + aggs[0][0] + aggs[0][1]
    else:
        a = jnp.concatenate(
            [hs[:, :H] + aggs[0][0] + aggs[0][1],
             hs[:, H:] + aggs[1][0] + aggs[1][1]], axis=1)
    t = jnp.maximum(
        jnp.dot(a, w1_ref[...], preferred_element_type=_f32) + b1_ref[...], 0.0)
    o = jnp.dot(t, w2_ref[...], preferred_element_type=_f32) + b2_ref[...]
    o_ref[...] = jnp.maximum(o, 0.0)


def _gin_layer(h, aggs, p):
    din = h.shape[1]
    scale = (1.0 + p["eps"]).reshape(1, 1).astype(_f32)
    return pl.pallas_call(
        functools.partial(_gin_mlp_kernel, len(aggs)),
        grid=(_NB,),
        in_specs=[
            pl.BlockSpec((_BN, din), lambda i: (i, 0)),
        ] + [
            pl.BlockSpec((2, _BN, H), lambda i: (0, i, 0)) for _ in aggs
        ] + [
            _full((1, 1)),
            _full((din, H)),
            _full((1, H)),
            _full((H, H)),
            _full((1, H)),
        ],
        out_specs=pl.BlockSpec((_BN, H), lambda i: (i, 0)),
        out_shape=jax.ShapeDtypeStruct((N, H), _f32),
    )(h, *aggs, scale, p["W1"], p["b1"].reshape(1, H), p["W2"],
      p["b2"].reshape(1, H))


def _gin_mlp4_kernel(nagg, *refs):
    hs = refs[:nagg]
    aggs = refs[nagg:2 * nagg]
    scale_ref, w1_ref, b1_ref, w2_ref, b2_ref, o_ref = refs[2 * nagg:]
    parts = [hs[i][0] * scale_ref[0, 0] + aggs[i][0, 0] + aggs[i][0, 1]
             for i in range(nagg)]
    a = parts[0] if nagg == 1 else jnp.concatenate(parts, axis=1)
    t = jnp.maximum(
        jnp.dot(a, w1_ref[...], preferred_element_type=_f32) + b1_ref[...], 0.0)
    o = jnp.dot(t, w2_ref[...], preferred_element_type=_f32) + b2_ref[...]
    o_ref[0] = jnp.maximum(o, 0.0)


def _gin_layer4(hc_parts, agg4s, p):
    nagg = len(hc_parts)
    din = H * nagg
    scale = (1.0 + p["eps"]).reshape(1, 1).astype(_f32)
    return pl.pallas_call(
        functools.partial(_gin_mlp4_kernel, nagg),
        grid=(NEXP, _NB),
        in_specs=[
            pl.BlockSpec((1, _BN, H), lambda e, i: (e, i, 0))
            for _ in hc_parts
        ] + [
            pl.BlockSpec((1, 2, _BN, H), lambda e, i: (e, 0, i, 0))
            for _ in agg4s
        ] + [
            _full((1, 1)),
            _full((din, H)),
            _full((1, H)),
            _full((H, H)),
            _full((1, H)),
        ],
        out_specs=pl.BlockSpec((1, _BN, H), lambda e, i: (e, i, 0)),
        out_shape=jax.ShapeDtypeStruct((NEXP, N, H), _f32),
    )(*hc_parts, *agg4s, scale, p["W1"], p["b1"].reshape(1, H), p["W2"],
      p["b2"].reshape(1, H))


def _prep_kernel(x_ref, h_ref, nw1, nb1, nw2, nb2, fw1, fb1, fw2, fb2,
                 xlo_ref, xhi_ref, nm_ref):
    h = h_ref[...]
    x = x_ref[...]
    for e in range(NEXP):
        t = jnp.maximum(
            jnp.dot(h, nw1[e], preferred_element_type=_f32) + nb1[e], 0.0)
        nm = jax.nn.sigmoid(
            jnp.dot(t, nw2[e], preferred_element_type=_f32) + nb2[e])
        t2 = jnp.maximum(
            jnp.dot(h, fw1[e], preferred_element_type=_f32) + fb1[e], 0.0)
        fm = jax.nn.sigmoid(
            jnp.dot(t2, fw2[e], preferred_element_type=_f32) + fb2[e])
        xm = x * fm * nm
        xlo_ref[e] = xm[:, :H]
        xhi_ref[e] = xm[:, H:]
        nm_ref[:, e:e + 1] = nm


def _prep(x, h, nmps, fmps):
    nw1 = jnp.stack([p["W1"] for p in nmps])
    nb1 = jnp.stack([p["b1"].reshape(1, H) for p in nmps])
    nw2 = jnp.stack([p["W2"] for p in nmps])
    nb2 = jnp.stack([p["b2"].reshape(1, 1) for p in nmps])
    fw1 = jnp.stack([p["W1"] for p in fmps])
    fb1 = jnp.stack([p["b1"].reshape(1, H) for p in fmps])
    fw2 = jnp.stack([p["W2"] for p in fmps])
    fb2 = jnp.stack([p["b2"].reshape(1, F) for p in fmps])
    return pl.pallas_call(
        _prep_kernel,
        grid=(_NB,),
        in_specs=[
            pl.BlockSpec((_BN, F), lambda i: (i, 0)),
            pl.BlockSpec((_BN, H), lambda i: (i, 0)),
            _full((NEXP, H, H)), _full((NEXP, 1, H)),
            _full((NEXP, H, 1)), _full((NEXP, 1, 1)),
            _full((NEXP, H, H)), _full((NEXP, 1, H)),
            _full((NEXP, H, F)), _full((NEXP, 1, F)),
        ],
        out_specs=[
            pl.BlockSpec((NEXP, _BN, H), lambda i: (0, i, 0)),
            pl.BlockSpec((NEXP, _BN, H), lambda i: (0, i, 0)),
            pl.BlockSpec((_BN, NEXP), lambda i: (i, 0)),
        ],
        out_shape=[
            jax.ShapeDtypeStruct((NEXP, N, H), _f32),
            jax.ShapeDtypeStruct((NEXP, N, H), _f32),
            jax.ShapeDtypeStruct((N, NEXP), _f32),
        ],
    )(x, h, nw1, nb1, nw2, nb2, fw1, fb1, fw2, fb2)


def _edge_mlp_kernel(hs_ref, hd_ref, w1a, w1b, b1, w2, b2, o_ref):
    hs = hs_ref[0]
    hd = hd_ref[0]
    for e in range(NEXP):
        z = jnp.maximum(
            jnp.dot(hs, w1a[e], preferred_element_type=_f32)
            + jnp.dot(hd, w1b[e], preferred_element_type=_f32) + b1[e], 0.0)
        lg = jnp.dot(z, w2[e], preferred_element_type=_f32) + b2[e]
        o_ref[:, e:e + 1] = jax.nn.sigmoid(lg)


def _edge_mlp(hsd, emps):
    w1a = jnp.stack([p["W1"][:H] for p in emps])
    w1b = jnp.stack([p["W1"][H:] for p in emps])
    b1 = jnp.stack([p["b1"].reshape(1, H) for p in emps])
    w2 = jnp.stack([p["W2"] for p in emps])
    b2 = jnp.stack([p["b2"].reshape(1, 1) for p in emps])
    return pl.pallas_call(
        _edge_mlp_kernel,
        grid=(E // _BE,),
        in_specs=[
            pl.BlockSpec((1, _BE, H), lambda i: (0, i, 0)),
            pl.BlockSpec((1, _BE, H), lambda i: (1, i, 0)),
            _full((NEXP, H, H)), _full((NEXP, H, H)), _full((NEXP, 1, H)),
            _full((NEXP, H, 1)), _full((NEXP, 1, 1)),
        ],
        out_specs=pl.BlockSpec((_BE, NEXP), lambda i: (i, 0)),
        out_shape=jax.ShapeDtypeStruct((E, NEXP), _f32),
    )(hsd, hsd, w1a, w1b, b1, w2, b2)


def _pool_kernel(hc_ref, nm_ref, b_ref, cw_ref, cb_ref, o_ref, gacc, cacc):
    i = pl.program_id(0)

    @pl.when(i == 0)
    def _():
        gacc[...] = jnp.zeros((NEXP, NGRAPH, H), _f32)
        cacc[...] = jnp.zeros((NGRAPH, NEXP), _f32)

    bb = b_ref[0, 0]
    oh = (bb[None, :] == lax.broadcasted_iota(_i32, (NGRAPH, _BN), 0)
          ).astype(_f32)
    nm = nm_ref[...]
    cacc[...] += jnp.dot(oh, nm, preferred_element_type=_f32)
    for e in range(NEXP):
        gacc[e] += jnp.dot(oh, hc_ref[e] * nm[:, e:e + 1],
                           preferred_element_type=_f32)

    @pl.when(i == _NB - 1)
    def _():
        for e in range(NEXP):
            g = gacc[e] / jnp.maximum(cacc[...][:, e:e + 1], 1e-6)
            o_ref[e] = (jnp.dot(g, cw_ref[e], preferred_element_type=_f32)
                        + cb_ref[e])


def _pool_clf(hc, nm4, batch, clfps):
    cw = jnp.stack([p["W"] for p in clfps])
    cb = jnp.stack([p["b"].reshape(1, NCLS) for p in clfps])
    b3 = batch.reshape(_NB, 1, _BN).astype(_i32)
    return pl.pallas_call(
        _pool_kernel,
        grid=(_NB,),
        in_specs=[
            pl.BlockSpec((NEXP, _BN, H), lambda i: (0, i, 0)),
            pl.BlockSpec((_BN, NEXP), lambda i: (i, 0)),
            pl.BlockSpec((1, 1, _BN), lambda i: (i, 0, 0)),
            _full((NEXP, H, NCLS)), _full((NEXP, 1, NCLS)),
        ],
        out_specs=pl.BlockSpec((NEXP, NGRAPH, NCLS), lambda i: (0, 0, 0)),
        out_shape=jax.ShapeDtypeStruct((NEXP, NGRAPH, NCLS), _f32),
        scratch_shapes=[
            pltpu.VMEM((NEXP, NGRAPH, H), _f32),
            pltpu.VMEM((NGRAPH, NEXP), _f32),
        ],
    )(hc, nm4, b3, cw, cb)


_make_segsum = functools.lru_cache(None)(_make_segsum)
_make_gather_rows = functools.lru_cache(None)(_make_gather_rows)


def _segsum_w4(h4, src, dst, w4):
    return _make_segsum(h4.shape[2], True, npass=NEXP)(h4, src, dst, w4)


def _segsum_unw(h, src, dst):
    return _make_segsum(h.shape[1], False)(h, src, dst)


def _gather64(h, edge_index):
    return _make_gather_rows(H)(h, edge_index)


def _segsum_cols(h, src, dst, w):
    def one(hh):
        if w is None:
            return _segsum_unw(hh, src, dst)
        return _segsum_w(hh, src, dst, w)

    if h.shape[1] == H:
        return [one(h)]
    return [one(h[:, :H]), one(h[:, H:])]


def kernel(x, edge_index, batch, params):
    src = edge_index[0]
    dst = edge_index[1]

    h = x
    for p in params["causal"]:
        aggs = _segsum_cols(h, src, dst, None)
        h = _gin_layer(h, aggs, p)

    xm_lo, xm_hi, nm4 = _prep(x, h, params["node_mask"], params["feat_mask"])

    hsd = _gather64(h, edge_index)
    em4 = _edge_mlp(hsd, params["edge_mask"]).T

    hc_parts = [xm_lo, xm_hi]
    for p in params["clf_enc"]:
        agg4s = [_segsum_w4(part, src, dst, em4) for part in hc_parts]
        hc_parts = [_gin_layer4(hc_parts, agg4s, p)]

    return _pool_clf(hc_parts[0], nm4, batch, params["clf"])
